# baseline (device time: 103892 ns/iter reference)
import jax
import jax.numpy as jnp
from jax import lax
from jax.experimental import pallas as pl
from jax.experimental.pallas import tpu as pltpu

N_DEV = 32
R_HOPS = N_DEV // 2
L_HOPS = N_DEV - 1 - R_HOPS
SPLIT = 4

def _mesh_index(x, y, z):
    return z * 8 + y * 2 + (x if y % 2 == 0 else 1 - x)

_RING_COORDS = (
    [(0, y, z) for y in range(4) for z in (range(4) if y % 2 == 0 else range(3, -1, -1))]
    + [(1, y, z) for y in range(3, -1, -1) for z in (range(4) if y % 2 == 1 else range(3, -1, -1))]
)
assert len(set(_RING_COORDS)) == N_DEV
for _p in range(N_DEV):
    _a, _b = _RING_COORDS[_p], _RING_COORDS[(_p + 1) % N_DEV]
    assert sum(abs(_a[i] - _b[i]) for i in range(3)) == 1, (_p, _a, _b)

_MESH_OF_RING = [_mesh_index(*c) for c in _RING_COORDS]
_RING_OF_MESH = [0] * N_DEV
for _p, _m in enumerate(_MESH_OF_RING):
    _RING_OF_MESH[_m] = _p


def kernel(x, w_mat, scale_x, scale_w):
    m_per, k = x.shape
    _, n = w_mat.shape
    m_total = N_DEV * m_per

    mesh_of_ring = jnp.asarray(_MESH_OF_RING, jnp.int32)
    ring_of_mesh = jnp.asarray(_RING_OF_MESH, jnp.int32)
    my_mesh = lax.axis_index("i")
    r = ring_of_mesh[my_mesh]
    nbrs = mesh_of_ring[jnp.stack([(r + 1) % N_DEV, (r - 1) % N_DEV])]
    r_orig = mesh_of_ring[(r - 1 - jnp.arange(R_HOPS)) % N_DEV]
    l_orig = mesh_of_ring[(r + 1 + jnp.arange(L_HOPS)) % N_DEV]

    def body(x_ref, w_ref, sx_ref, sw_ref, nbrs_ref, rorig_ref, lorig_ref,
             out_ref, rbuf, lbuf, w_bf16_ref,
             r_send, r_recv, l_send, l_recv):
        right = nbrs_ref[0]
        left = nbrs_ref[1]

        barrier_sem = pltpu.get_barrier_semaphore()
        for nbr in (left, right):
            pl.semaphore_signal(
                barrier_sem, inc=1,
                device_id=(nbr,), device_id_type=pl.DeviceIdType.MESH,
            )
        pl.semaphore_wait(barrier_sem, 2)

        s = sx_ref[0] * sw_ref[0]
        rows = m_per // SPLIT

        def rsend(h, j):
            c = pltpu.make_async_remote_copy(
                src_ref=rbuf.at[h, pl.ds(j * rows, rows)],
                dst_ref=rbuf.at[h + 1, pl.ds(j * rows, rows)],
                send_sem=r_send.at[h * SPLIT + j],
                recv_sem=r_recv.at[h * SPLIT + j],
                device_id=(right,), device_id_type=pl.DeviceIdType.MESH,
            )
            c.start()
            return c

        def lsend(h, j):
            c = pltpu.make_async_remote_copy(
                src_ref=lbuf.at[h, pl.ds(j * rows, rows)],
                dst_ref=lbuf.at[h + 1, pl.ds(j * rows, rows)],
                send_sem=l_send.at[h * SPLIT + j],
                recv_sem=l_recv.at[h * SPLIT + j],
                device_id=(left,), device_id_type=pl.DeviceIdType.MESH,
            )
            c.start()
            return c

        def gemm(src_ref, origin):
            out_ref[pl.ds(origin * m_per, m_per), :] = (
                jnp.dot(src_ref[...].astype(jnp.bfloat16), w_bf16_ref[...],
                        preferred_element_type=jnp.float32) * s
            )

        x8 = x_ref[...].astype(jnp.float8_e4m3fn)
        rbuf[0] = x8
        lbuf[0] = x8
        rdmas_r = [rsend(0, j) for j in range(SPLIT)]
        rdmas_l = [lsend(0, j) for j in range(SPLIT)]

        w_bf16_ref[...] = w_ref[...].astype(jnp.bfloat16)
        gemm(x_ref, lax.axis_index("i"))

        for h in range(R_HOPS):
            next_r = [None] * SPLIT
            next_l = [None] * SPLIT
            for j in range(SPLIT):
                rdmas_r[j].wait_recv()
                if h + 1 < R_HOPS:
                    next_r[j] = rsend(h + 1, j)
                if h < L_HOPS:
                    rdmas_l[j].wait_recv()
                    if h + 1 < L_HOPS:
                        next_l[j] = lsend(h + 1, j)
            gemm(rbuf.at[h + 1], rorig_ref[h])
            if h < L_HOPS:
                gemm(lbuf.at[h + 1], lorig_ref[h])
                for j in range(SPLIT):
                    rdmas_l[j].wait_send()
                rdmas_l = next_l
            for j in range(SPLIT):
                rdmas_r[j].wait_send()
            rdmas_r = next_r

    return pl.pallas_call(
        body,
        out_shape=jax.ShapeDtypeStruct((m_total, n), jnp.float32),
        in_specs=[
            pl.BlockSpec(memory_space=pltpu.VMEM),
            pl.BlockSpec(memory_space=pltpu.VMEM),
            pl.BlockSpec(memory_space=pltpu.SMEM),
            pl.BlockSpec(memory_space=pltpu.SMEM),
            pl.BlockSpec(memory_space=pltpu.SMEM),
            pl.BlockSpec(memory_space=pltpu.SMEM),
            pl.BlockSpec(memory_space=pltpu.SMEM),
        ],
        out_specs=pl.BlockSpec(memory_space=pltpu.VMEM),
        scratch_shapes=[
            pltpu.VMEM((R_HOPS + 1, m_per, k), jnp.float8_e4m3fn),
            pltpu.VMEM((L_HOPS + 1, m_per, k), jnp.float8_e4m3fn),
            pltpu.VMEM((k, n), jnp.bfloat16),
            pltpu.SemaphoreType.DMA((R_HOPS * SPLIT,)),
            pltpu.SemaphoreType.DMA((R_HOPS * SPLIT,)),
            pltpu.SemaphoreType.DMA((L_HOPS * SPLIT,)),
            pltpu.SemaphoreType.DMA((L_HOPS * SPLIT,)),
        ],
        compiler_params=pltpu.CompilerParams(collective_id=0),
    )(x, w_mat, scale_x, scale_w, nbrs, r_orig, l_orig)


# device time: 102599 ns/iter; 1.0126x vs baseline; 1.0126x over previous
import jax
import jax.numpy as jnp
from jax import lax
from jax.experimental import pallas as pl
from jax.experimental.pallas import tpu as pltpu

N_DEV = 32
HOPS = N_DEV // 2
SPLIT = 2

def _mesh_index(x, y, z):
    return z * 8 + y * 2 + (x if y % 2 == 0 else 1 - x)

_RING_COORDS = (
    [(0, y, z) for y in range(4) for z in (range(4) if y % 2 == 0 else range(3, -1, -1))]
    + [(1, y, z) for y in range(3, -1, -1) for z in (range(4) if y % 2 == 1 else range(3, -1, -1))]
)
assert len(set(_RING_COORDS)) == N_DEV
for _p in range(N_DEV):
    _a, _b = _RING_COORDS[_p], _RING_COORDS[(_p + 1) % N_DEV]
    assert sum(abs(_a[i] - _b[i]) for i in range(3)) == 1, (_p, _a, _b)

_MESH_OF_RING = [_mesh_index(*c) for c in _RING_COORDS]
_RING_OF_MESH = [0] * N_DEV
for _p, _m in enumerate(_MESH_OF_RING):
    _RING_OF_MESH[_m] = _p


def kernel(x, w_mat, scale_x, scale_w):
    m_per, k = x.shape
    _, n = w_mat.shape
    m_total = N_DEV * m_per
    rows = m_per // SPLIT

    mesh_of_ring = jnp.asarray(_MESH_OF_RING, jnp.int32)
    ring_of_mesh = jnp.asarray(_RING_OF_MESH, jnp.int32)
    my_mesh = lax.axis_index("i")
    r = ring_of_mesh[my_mesh]
    nbrs = mesh_of_ring[jnp.stack([(r + 1) % N_DEV, (r - 1) % N_DEV])]
    r_orig = mesh_of_ring[(r - 1 - jnp.arange(HOPS)) % N_DEV]
    l_orig = mesh_of_ring[(r + 1 + jnp.arange(HOPS)) % N_DEV]

    def body(x_ref, w_ref, sx_ref, sw_ref, nbrs_ref, rorig_ref, lorig_ref,
             out_ref, rbuf, lbuf, w_bf16_ref,
             r_send, r_recv, l_send, l_recv):
        right = nbrs_ref[0]
        left = nbrs_ref[1]

        barrier_sem = pltpu.get_barrier_semaphore()
        for nbr in (left, right):
            pl.semaphore_signal(
                barrier_sem, inc=1,
                device_id=(nbr,), device_id_type=pl.DeviceIdType.MESH,
            )
        pl.semaphore_wait(barrier_sem, 2)

        s = sx_ref[0] * sw_ref[0]

        def rjs(h):
            return (0,) if h == HOPS - 1 else tuple(range(SPLIT))

        def ljs(h):
            return (SPLIT - 1,) if h == HOPS - 1 else tuple(range(SPLIT))

        def rsend(h, j):
            c = pltpu.make_async_remote_copy(
                src_ref=rbuf.at[h, pl.ds(j * rows, rows)],
                dst_ref=rbuf.at[h + 1, pl.ds(j * rows, rows)],
                send_sem=r_send.at[h * SPLIT + j],
                recv_sem=r_recv.at[h * SPLIT + j],
                device_id=(right,), device_id_type=pl.DeviceIdType.MESH,
            )
            c.start()
            return c

        def lsend(h, j):
            src = rbuf if h == 0 else lbuf
            c = pltpu.make_async_remote_copy(
                src_ref=src.at[h, pl.ds(j * rows, rows)],
                dst_ref=lbuf.at[h + 1, pl.ds(j * rows, rows)],
                send_sem=l_send.at[h * SPLIT + j],
                recv_sem=l_recv.at[h * SPLIT + j],
                device_id=(left,), device_id_type=pl.DeviceIdType.MESH,
            )
            c.start()
            return c

        def gemm(src_ref, origin, row0, nrows):
            out_ref[pl.ds(origin * m_per + row0, nrows), :] = (
                jnp.dot(src_ref[pl.ds(row0, nrows), :].astype(jnp.bfloat16),
                        w_bf16_ref[...],
                        preferred_element_type=jnp.float32) * s
            )

        rbuf[0] = x_ref[...].astype(jnp.float8_e4m3fn)
        rdmas_r = {j: rsend(0, j) for j in rjs(0)}
        rdmas_l = {j: lsend(0, j) for j in ljs(0)}

        w_bf16_ref[...] = w_ref[...].astype(jnp.bfloat16)
        gemm(x_ref, lax.axis_index("i"), 0, m_per)

        for h in range(HOPS):
            next_r, next_l = {}, {}
            for j in range(SPLIT):
                if j in rdmas_r:
                    rdmas_r[j].wait_recv()
                    if h + 1 < HOPS and j in rjs(h + 1):
                        next_r[j] = rsend(h + 1, j)
                if j in rdmas_l:
                    rdmas_l[j].wait_recv()
                    if h + 1 < HOPS and j in ljs(h + 1):
                        next_l[j] = lsend(h + 1, j)
            if h == HOPS - 1:
                gemm(rbuf.at[h + 1], rorig_ref[h], 0, rows)
                gemm(lbuf.at[h + 1], lorig_ref[h], (SPLIT - 1) * rows, rows)
            else:
                gemm(rbuf.at[h + 1], rorig_ref[h], 0, m_per)
                gemm(lbuf.at[h + 1], lorig_ref[h], 0, m_per)
            for c in rdmas_l.values():
                c.wait_send()
            for c in rdmas_r.values():
                c.wait_send()
            rdmas_r, rdmas_l = next_r, next_l

    return pl.pallas_call(
        body,
        out_shape=jax.ShapeDtypeStruct((m_total, n), jnp.float32),
        in_specs=[
            pl.BlockSpec(memory_space=pltpu.VMEM),
            pl.BlockSpec(memory_space=pltpu.VMEM),
            pl.BlockSpec(memory_space=pltpu.SMEM),
            pl.BlockSpec(memory_space=pltpu.SMEM),
            pl.BlockSpec(memory_space=pltpu.SMEM),
            pl.BlockSpec(memory_space=pltpu.SMEM),
            pl.BlockSpec(memory_space=pltpu.SMEM),
        ],
        out_specs=pl.BlockSpec(memory_space=pltpu.VMEM),
        scratch_shapes=[
            pltpu.VMEM((HOPS + 1, m_per, k), jnp.float8_e4m3fn),
            pltpu.VMEM((HOPS + 1, m_per, k), jnp.float8_e4m3fn),
            pltpu.VMEM((k, n), jnp.bfloat16),
            pltpu.SemaphoreType.DMA((HOPS * SPLIT,)),
            pltpu.SemaphoreType.DMA((HOPS * SPLIT,)),
            pltpu.SemaphoreType.DMA((HOPS * SPLIT,)),
            pltpu.SemaphoreType.DMA((HOPS * SPLIT,)),
        ],
        compiler_params=pltpu.CompilerParams(collective_id=0),
    )(x, w_mat, scale_x, scale_w, nbrs, r_orig, l_orig)


# device time: 100617 ns/iter; 1.0325x vs baseline; 1.0197x over previous
import jax
import jax.numpy as jnp
from jax import lax
from jax.experimental import pallas as pl
from jax.experimental.pallas import tpu as pltpu

N_DEV = 32
HOPS = N_DEV // 2
SPLIT = 2

def _mesh_index(x, y, z):
    return z * 8 + y * 2 + (x if y % 2 == 0 else 1 - x)

_RING_COORDS = (
    [(0, y, z) for y in range(4) for z in (range(4) if y % 2 == 0 else range(3, -1, -1))]
    + [(1, y, z) for y in range(3, -1, -1) for z in (range(4) if y % 2 == 1 else range(3, -1, -1))]
)
assert len(set(_RING_COORDS)) == N_DEV
for _p in range(N_DEV):
    _a, _b = _RING_COORDS[_p], _RING_COORDS[(_p + 1) % N_DEV]
    assert sum(abs(_a[i] - _b[i]) for i in range(3)) == 1, (_p, _a, _b)

_MESH_OF_RING = [_mesh_index(*c) for c in _RING_COORDS]
_RING_OF_MESH = [0] * N_DEV
for _p, _m in enumerate(_MESH_OF_RING):
    _RING_OF_MESH[_m] = _p

_HOPS = N_DEV // 2
_TABLE = []
for _m in range(N_DEV):
    _r = _RING_OF_MESH[_m]
    _row = [_MESH_OF_RING[(_r + 1) % N_DEV], _MESH_OF_RING[(_r - 1) % N_DEV]]
    _row += [_MESH_OF_RING[(_r - 1 - _h) % N_DEV] for _h in range(_HOPS)]
    _row += [_MESH_OF_RING[(_r + 1 + _h) % N_DEV] for _h in range(_HOPS)]
    _TABLE.append(_row)


def kernel(x, w_mat, scale_x, scale_w):
    m_per, k = x.shape
    _, n = w_mat.shape
    m_total = N_DEV * m_per
    rows = m_per // SPLIT

    row = jnp.asarray(_TABLE, jnp.int32)[lax.axis_index("i")]
    nbrs = row[0:2]
    r_orig = row[2:2 + HOPS]
    l_orig = row[2 + HOPS:2 + 2 * HOPS]

    def body(x_ref, w_ref, sx_ref, sw_ref, nbrs_ref, rorig_ref, lorig_ref,
             out_ref, rbuf, lbuf, w_bf16_ref,
             r_send, r_recv, l_send, l_recv):
        right = nbrs_ref[0]
        left = nbrs_ref[1]

        rbuf[0] = x_ref[...].astype(jnp.float8_e4m3fn)

        barrier_sem = pltpu.get_barrier_semaphore()
        for nbr in (left, right):
            pl.semaphore_signal(
                barrier_sem, inc=1,
                device_id=(nbr,), device_id_type=pl.DeviceIdType.MESH,
            )
        pl.semaphore_wait(barrier_sem, 2)

        s = sx_ref[0] * sw_ref[0]

        def rjs(h):
            return (0,) if h == HOPS - 1 else tuple(range(SPLIT))

        def ljs(h):
            return (SPLIT - 1,) if h == HOPS - 1 else tuple(range(SPLIT))

        def rsend(h, j):
            c = pltpu.make_async_remote_copy(
                src_ref=rbuf.at[h, pl.ds(j * rows, rows)],
                dst_ref=rbuf.at[h + 1, pl.ds(j * rows, rows)],
                send_sem=r_send.at[h * SPLIT + j],
                recv_sem=r_recv.at[h * SPLIT + j],
                device_id=(right,), device_id_type=pl.DeviceIdType.MESH,
            )
            c.start()
            return c

        def lsend(h, j):
            src = rbuf if h == 0 else lbuf
            c = pltpu.make_async_remote_copy(
                src_ref=src.at[h, pl.ds(j * rows, rows)],
                dst_ref=lbuf.at[h + 1, pl.ds(j * rows, rows)],
                send_sem=l_send.at[h * SPLIT + j],
                recv_sem=l_recv.at[h * SPLIT + j],
                device_id=(left,), device_id_type=pl.DeviceIdType.MESH,
            )
            c.start()
            return c

        def gemm(src_ref, origin, row0, nrows):
            out_ref[pl.ds(origin * m_per + row0, nrows), :] = (
                jnp.dot(src_ref[pl.ds(row0, nrows), :].astype(jnp.bfloat16),
                        w_bf16_ref[...],
                        preferred_element_type=jnp.float32) * s
            )

        rdmas_r = {j: rsend(0, j) for j in rjs(0)}
        rdmas_l = {j: lsend(0, j) for j in ljs(0)}

        w_bf16_ref[...] = w_ref[...].astype(jnp.bfloat16)
        gemm(x_ref, lax.axis_index("i"), 0, m_per)

        for h in range(HOPS):
            next_r, next_l = {}, {}
            for j in range(SPLIT):
                if j in rdmas_r:
                    rdmas_r[j].wait_recv()
                    if h + 1 < HOPS and j in rjs(h + 1):
                        next_r[j] = rsend(h + 1, j)
                if j in rdmas_l:
                    rdmas_l[j].wait_recv()
                    if h + 1 < HOPS and j in ljs(h + 1):
                        next_l[j] = lsend(h + 1, j)
            if h == HOPS - 1:
                gemm(rbuf.at[h + 1], rorig_ref[h], 0, rows)
                gemm(lbuf.at[h + 1], lorig_ref[h], (SPLIT - 1) * rows, rows)
            else:
                gemm(rbuf.at[h + 1], rorig_ref[h], 0, m_per)
                gemm(lbuf.at[h + 1], lorig_ref[h], 0, m_per)
            for c in rdmas_l.values():
                c.wait_send()
            for c in rdmas_r.values():
                c.wait_send()
            rdmas_r, rdmas_l = next_r, next_l

    return pl.pallas_call(
        body,
        out_shape=jax.ShapeDtypeStruct((m_total, n), jnp.float32),
        in_specs=[
            pl.BlockSpec(memory_space=pltpu.VMEM),
            pl.BlockSpec(memory_space=pltpu.VMEM),
            pl.BlockSpec(memory_space=pltpu.SMEM),
            pl.BlockSpec(memory_space=pltpu.SMEM),
            pl.BlockSpec(memory_space=pltpu.SMEM),
            pl.BlockSpec(memory_space=pltpu.SMEM),
            pl.BlockSpec(memory_space=pltpu.SMEM),
        ],
        out_specs=pl.BlockSpec(memory_space=pltpu.VMEM),
        scratch_shapes=[
            pltpu.VMEM((HOPS + 1, m_per, k), jnp.float8_e4m3fn),
            pltpu.VMEM((HOPS + 1, m_per, k), jnp.float8_e4m3fn),
            pltpu.VMEM((k, n), jnp.bfloat16),
            pltpu.SemaphoreType.DMA((HOPS * SPLIT,)),
            pltpu.SemaphoreType.DMA((HOPS * SPLIT,)),
            pltpu.SemaphoreType.DMA((HOPS * SPLIT,)),
            pltpu.SemaphoreType.DMA((HOPS * SPLIT,)),
        ],
        compiler_params=pltpu.CompilerParams(collective_id=0),
    )(x, w_mat, scale_x, scale_w, nbrs, r_orig, l_orig)
